# baseline (speedup 1.0000x reference)
import functools

import jax
import jax.numpy as jnp
from jax import lax
from jax.experimental import pallas as pl
from jax.experimental.pallas import tpu as pltpu
from jax.experimental.pallas import tpu_sc as plsc

EMBED = 128
VOCAB = 1024
SEQ = 200
SEQ_PAD = 208
NCHUNK = 13
NC, NS = 2, 16
NW = NC * NS
BATCH = 1024
ROWS_PER_W = BATCH // NW
ROW_ELEMS = SEQ * EMBED

ZR = 3 * VOCAB
TBL_ROWS = 3 * VOCAB + 8

SCALE = 1024.0
QMAX = 10922
BIAS = 10922

G_LO, G_HI = 128, SEQ - 128


def _sc_body(pos_hbm, tbl_hbm, eos_hbm, out_hbm,
             pos_v0, pos_v1, idx_v0, idx_v1, acc_v0, acc_v1, f32_v0, f32_v1,
             eos_v, sem_g0, sem_add, sem_out, sem_pos):
    wid = lax.axis_index("s") * NC + lax.axis_index("c")
    lane = lax.iota(jnp.int32, 16)
    base_row = wid * ROWS_PER_W

    def fire_pos(i, pos_v):
        pltpu.async_copy(pos_hbm.at[pl.ds((base_row + i) * (SEQ * 3), SEQ * 3)],
                         pos_v.at[pl.ds(0, SEQ * 3)], sem_pos)

    def wait_pos(pos_v):
        pltpu.make_async_copy(pos_hbm.at[pl.ds(0, SEQ * 3)],
                              pos_v.at[pl.ds(0, SEQ * 3)], sem_pos).wait()

    def compute_idx(pos_v, idx_v):
        def chunk(c, carry):
            tb = pl.multiple_of(c * 16, 16)
            tvec = tb + lane
            base3 = tvec * 3
            for a in range(3):
                pv = plsc.load_gather(pos_v, [base3 + a])
                idx = jnp.where(pv == 0, ZR + a, pv + a * VOCAB)
                idx_v[a, pl.ds(tb, 16)] = idx
            return carry

        lax.fori_loop(0, NCHUNK, chunk, 0)

    def g0_copies(idx_v, acc_v, make_only):
        mk = pltpu.make_async_copy if make_only else (
            lambda s, d, m: pltpu.async_copy(s, d, m))
        return [
            mk(tbl_hbm.at[idx_v.at[0, pl.ds(0, G_LO)]],
               acc_v.at[pl.ds(0, G_LO)], sem_g0),
            mk(tbl_hbm.at[idx_v.at[0, pl.ds(G_LO, G_HI)]],
               acc_v.at[pl.ds(G_LO, G_HI)], sem_g0),
        ]

    def add_copies(idx_v, acc_v, make_only):
        cps = []
        for k in range(1, 3):
            for off, ln in ((0, G_LO), (G_LO, G_HI)):
                src = tbl_hbm.at[idx_v.at[k, pl.ds(off, ln)]]
                dst = acc_v.at[pl.ds(off, ln)]
                if make_only:
                    cps.append(pltpu.make_async_copy(src, dst, sem_add))
                else:
                    cps.append(pltpu.async_copy(src, dst, sem_add, add=True))
        return cps

    def combine(acc_v, f32_v):
        inv = jnp.float32(1.0 / SCALE)
        off6 = jnp.float32(6 * BIAS)
        off3 = jnp.float32(3 * BIAS)
        for h in range(EMBED // 16):
            col = 16 * h
            w0 = acc_v[0, pl.ds(col, 16)]

            def tok(t, w_prev):
                w_t = acc_v[t, pl.ds(col, 16)]
                s = (w_prev & 0xFFFF) + lax.shift_right_logical(w_t, 16)
                f32_v[t - 1, pl.ds(col, 16)] = (
                    (s.astype(jnp.float32) - off6) * inv)
                return w_t

            w_last = lax.fori_loop(1, SEQ, tok, w0)
            ev = eos_v[pl.ds(col, 16)]
            f32_v[SEQ - 1, pl.ds(col, 16)] = (
                ((w_last & 0xFFFF).astype(jnp.float32) - off3) * inv + ev)

    def fire_out(i, f32_v):
        pltpu.async_copy(f32_v, out_hbm.at[base_row + i], sem_out)

    def wait_out(i, f32_v):
        pltpu.make_async_copy(f32_v, out_hbm.at[base_row + i], sem_out).wait()

    pos_b = (pos_v0, pos_v1)
    idx_b = (idx_v0, idx_v1)
    acc_b = (acc_v0, acc_v1)
    f32_b = (f32_v0, f32_v1)

    pltpu.sync_copy(eos_hbm, eos_v)
    pltpu.sync_copy(pos_hbm.at[pl.ds(base_row * (SEQ * 3), SEQ * 3)],
                    pos_v0.at[pl.ds(0, SEQ * 3)])
    compute_idx(pos_v0, idx_v0)
    fire_pos(1, pos_v1)

    def stage(i, p):
        pos_c, pos_n = pos_b[p], pos_b[1 - p]
        idx_c, idx_n = idx_b[p], idx_b[1 - p]
        acc_c, acc_p = acc_b[p], acc_b[1 - p]
        f32_c, f32_p = f32_b[p], f32_b[1 - p]

        @pl.when(i >= 2)
        def _():
            wait_out(i - 2, f32_c)

        @pl.when(i <= ROWS_PER_W - 1)
        def _():
            g0_copies(idx_c, acc_c, False)

        @pl.when(jnp.logical_and(i >= 1, i <= ROWS_PER_W))
        def _():
            for cp in add_copies(idx_n, acc_p, True):
                cp.wait()

        @pl.when(i <= ROWS_PER_W - 1)
        def _():
            for cp in g0_copies(idx_c, acc_c, True):
                cp.wait()
            add_copies(idx_c, acc_c, False)

        @pl.when(jnp.logical_and(i >= 1, i <= ROWS_PER_W))
        def _():
            combine(acc_p, f32_p)
            fire_out(i - 1, f32_p)

        @pl.when(i <= ROWS_PER_W - 2)
        def _():
            wait_pos(pos_n)

        @pl.when(i <= ROWS_PER_W - 3)
        def _():
            fire_pos(i + 2, pos_c)

        @pl.when(i <= ROWS_PER_W - 2)
        def _():
            compute_idx(pos_n, idx_n)

    def pair(j, carry):
        stage(2 * j, 0)
        stage(2 * j + 1, 1)
        return carry

    lax.fori_loop(0, (ROWS_PER_W + 2) // 2, pair, 0)


def _encode(w):
    q = jnp.clip(jnp.round(w * SCALE), -QMAX, QMAX).astype(jnp.int32)
    return q + BIAS


def kernel(position, W_spatial, W_la, eos):
    n, s, a = position.shape
    assert (n, s, a) == (BATCH, SEQ, 3)
    pos2 = position.reshape(n * s * a)
    lo = jnp.concatenate([_encode(W_spatial.reshape(3 * VOCAB, EMBED)),
                          jnp.full((TBL_ROWS - 3 * VOCAB, EMBED), BIAS,
                                   jnp.int32)], axis=0)
    hi = jnp.concatenate([_encode(W_la.reshape(3 * VOCAB, EMBED)),
                          jnp.full((TBL_ROWS - 3 * VOCAB, EMBED), BIAS,
                                   jnp.int32)], axis=0)
    tbl = lo | (hi << 16)

    mesh = plsc.VectorSubcoreMesh(core_axis_name="c", subcore_axis_name="s")
    run = pl.kernel(
        _sc_body,
        out_type=jax.ShapeDtypeStruct((BATCH, SEQ, EMBED), jnp.float32),
        mesh=mesh,
        scratch_types=[
            pltpu.VMEM((640,), jnp.int32),
            pltpu.VMEM((640,), jnp.int32),
            pltpu.VMEM((6, SEQ_PAD), jnp.int32),
            pltpu.VMEM((6, SEQ_PAD), jnp.int32),
            pltpu.VMEM((SEQ, EMBED), jnp.int32),
            pltpu.VMEM((SEQ, EMBED), jnp.int32),
            pltpu.VMEM((SEQ, EMBED), jnp.float32),
            pltpu.VMEM((SEQ, EMBED), jnp.float32),
            pltpu.VMEM((EMBED,), jnp.float32),
            pltpu.SemaphoreType.DMA,
            pltpu.SemaphoreType.DMA,
            pltpu.SemaphoreType.DMA,
            pltpu.SemaphoreType.DMA,
        ],
        compiler_params=pltpu.CompilerParams(needs_layout_passes=False),
    )
    return run(pos2, tbl, eos)


# speedup vs baseline: 1.3926x; 1.3926x over previous
import functools

import jax
import jax.numpy as jnp
from jax import lax
from jax.experimental import pallas as pl
from jax.experimental.pallas import tpu as pltpu
from jax.experimental.pallas import tpu_sc as plsc

EMBED = 128
VOCAB = 1024
SEQ = 200
SEQ_PAD = 208
NCHUNK = 13
NC, NS = 2, 16
NW = NC * NS
BATCH = 1024
ROWS_PER_W = BATCH // NW
NH = EMBED // 16

ZR = 3 * VOCAB
TBL_ROWS = 3 * VOCAB + 8

SCALE = 1024.0
QMAX = 10922
BIAS = 10922

G_LO, G_HI = 128, SEQ - 128


def _sc_body(pos_hbm, tbl_hbm, eos_hbm, out_hbm,
             pos_v0, pos_v1, idx_v0, idx_v1, acc_v0, acc_v1, f32_v0, f32_v1,
             eos_v, sem_add, sem_out, sem_pos):
    wid = lax.axis_index("s") * NC + lax.axis_index("c")
    lane = lax.iota(jnp.int32, 16)
    base_row = wid * ROWS_PER_W

    def fire_pos(i, pos_v):
        pltpu.async_copy(pos_hbm.at[pl.ds((base_row + i) * (SEQ * 3), SEQ * 3)],
                         pos_v.at[pl.ds(0, SEQ * 3)], sem_pos)

    def wait_pos(pos_v):
        pltpu.make_async_copy(pos_hbm.at[pl.ds(0, SEQ * 3)],
                              pos_v.at[pl.ds(0, SEQ * 3)], sem_pos).wait()

    def compute_idx(pos_v, idx_v):
        def chunk(c, carry):
            tb = pl.multiple_of(c * 16, 16)
            tvec = tb + lane
            base3 = tvec * 3
            for a in range(3):
                pv = plsc.load_gather(pos_v, [base3 + a])
                idx = jnp.where(pv == 0, ZR + a, pv + a * VOCAB)
                idx_v[a, pl.ds(tb, 16)] = idx
            return carry

        lax.fori_loop(0, NCHUNK, chunk, 0)

    def gather_copies(idx_v, acc_v, make_only):
        cps = []
        for k in range(3):
            for off, ln in ((0, G_LO), (G_LO, G_HI)):
                src = tbl_hbm.at[idx_v.at[k, pl.ds(off, ln)]]
                dst = acc_v.at[pl.ds(off, ln)]
                if make_only:
                    cps.append(pltpu.make_async_copy(src, dst, sem_add))
                else:
                    cps.append(pltpu.async_copy(src, dst, sem_add, add=True))
        return cps

    def combine(acc_v, f32_v):
        inv = jnp.float32(1.0 / SCALE)
        off6 = jnp.float32(6 * BIAS)
        off3 = jnp.float32(3 * BIAS)
        w0 = tuple(acc_v[0, pl.ds(16 * h, 16)] for h in range(NH))

        def tok(t, carry):
            nxt = []
            for h in range(NH):
                w_t = acc_v[t, pl.ds(16 * h, 16)]
                s = (carry[h] & 0xFFFF) + lax.shift_right_logical(w_t, 16)
                f32_v[t - 1, pl.ds(16 * h, 16)] = (
                    (s.astype(jnp.float32) - off6) * inv)
                nxt.append(w_t)
            return tuple(nxt)

        w_last = lax.fori_loop(1, SEQ, tok, w0)
        for h in range(NH):
            ev = eos_v[pl.ds(16 * h, 16)]
            f32_v[SEQ - 1, pl.ds(16 * h, 16)] = (
                ((w_last[h] & 0xFFFF).astype(jnp.float32) - off3) * inv + ev)

    def zero_acc(acc_v):
        z = jnp.zeros((16,), jnp.int32)

        def row(t, carry):
            for h in range(NH):
                acc_v[t, pl.ds(16 * h, 16)] = z
            return carry

        lax.fori_loop(0, SEQ, row, 0)

    def fire_out(i, f32_v):
        pltpu.async_copy(f32_v, out_hbm.at[base_row + i], sem_out)

    def wait_out(i, f32_v):
        pltpu.make_async_copy(f32_v, out_hbm.at[base_row + i], sem_out).wait()

    pos_b = (pos_v0, pos_v1)
    idx_b = (idx_v0, idx_v1)
    acc_b = (acc_v0, acc_v1)
    f32_b = (f32_v0, f32_v1)

    zero_acc(acc_v0)
    zero_acc(acc_v1)
    pltpu.sync_copy(eos_hbm, eos_v)
    pltpu.sync_copy(pos_hbm.at[pl.ds(base_row * (SEQ * 3), SEQ * 3)],
                    pos_v0.at[pl.ds(0, SEQ * 3)])
    compute_idx(pos_v0, idx_v0)
    fire_pos(1, pos_v1)

    def stage(i, p):
        pos_c, pos_n = pos_b[p], pos_b[1 - p]
        idx_c, idx_n = idx_b[p], idx_b[1 - p]
        acc_c, acc_p = acc_b[p], acc_b[1 - p]
        f32_c, f32_p = f32_b[p], f32_b[1 - p]

        @pl.when(i >= 2)
        def _():
            wait_out(i - 2, f32_c)

        @pl.when(i <= ROWS_PER_W - 1)
        def _():
            gather_copies(idx_c, acc_c, False)

        @pl.when(jnp.logical_and(i >= 1, i <= ROWS_PER_W))
        def _():
            for cp in gather_copies(idx_n, acc_p, True):
                cp.wait()

        @pl.when(jnp.logical_and(i >= 1, i <= ROWS_PER_W))
        def _():
            combine(acc_p, f32_p)
            fire_out(i - 1, f32_p)
            zero_acc(acc_p)

        @pl.when(i <= ROWS_PER_W - 2)
        def _():
            wait_pos(pos_n)

        @pl.when(i <= ROWS_PER_W - 3)
        def _():
            fire_pos(i + 2, pos_c)

        @pl.when(i <= ROWS_PER_W - 2)
        def _():
            compute_idx(pos_n, idx_n)

    def pair(j, carry):
        stage(2 * j, 0)
        stage(2 * j + 1, 1)
        return carry

    lax.fori_loop(0, (ROWS_PER_W + 2) // 2, pair, 0)


def _encode(w):
    q = jnp.clip(jnp.round(w * SCALE), -QMAX, QMAX).astype(jnp.int32)
    return q + BIAS


def kernel(position, W_spatial, W_la, eos):
    n, s, a = position.shape
    assert (n, s, a) == (BATCH, SEQ, 3)
    pos2 = position.reshape(n * s * a)
    lo = jnp.concatenate([_encode(W_spatial.reshape(3 * VOCAB, EMBED)),
                          jnp.full((TBL_ROWS - 3 * VOCAB, EMBED), BIAS,
                                   jnp.int32)], axis=0)
    hi = jnp.concatenate([_encode(W_la.reshape(3 * VOCAB, EMBED)),
                          jnp.full((TBL_ROWS - 3 * VOCAB, EMBED), BIAS,
                                   jnp.int32)], axis=0)
    tbl = lo | (hi << 16)

    mesh = plsc.VectorSubcoreMesh(core_axis_name="c", subcore_axis_name="s")
    run = pl.kernel(
        _sc_body,
        out_type=jax.ShapeDtypeStruct((BATCH, SEQ, EMBED), jnp.float32),
        mesh=mesh,
        scratch_types=[
            pltpu.VMEM((640,), jnp.int32),
            pltpu.VMEM((640,), jnp.int32),
            pltpu.VMEM((6, SEQ_PAD), jnp.int32),
            pltpu.VMEM((6, SEQ_PAD), jnp.int32),
            pltpu.VMEM((SEQ, EMBED), jnp.int32),
            pltpu.VMEM((SEQ, EMBED), jnp.int32),
            pltpu.VMEM((SEQ, EMBED), jnp.float32),
            pltpu.VMEM((SEQ, EMBED), jnp.float32),
            pltpu.VMEM((EMBED,), jnp.float32),
            pltpu.SemaphoreType.DMA,
            pltpu.SemaphoreType.DMA,
            pltpu.SemaphoreType.DMA,
        ],
        compiler_params=pltpu.CompilerParams(needs_layout_passes=False),
    )
    return run(pos2, tbl, eos)
